# baseline (speedup 1.0000x reference)
import jax
import jax.numpy as jnp
from jax import lax
from jax.experimental import pallas as pl
from jax.experimental.pallas import tpu as pltpu
from jax.experimental.pallas import tpu_sc as plsc

_DIM = 256
_HEADS = 8
_HD = _DIM // _HEADS
_N = 10000
_E = 160000

_NP = 10240
_TILES = 16
_CHUNK = 80
_EPT = 10240
_NCHUNK = _EPT // _CHUNK
_EPAD = _TILES * _EPT


def _prep_body(x_ref, wt_ref, b_ref, A_ref, hlo_ref, hhi_ref, stab_ref):
  h = jnp.dot(x_ref[...], wt_ref[...], preferred_element_type=jnp.float32)
  h = h + b_ref[...]
  st = jnp.dot(h, A_ref[...], preferred_element_type=jnp.float32)
  hlo_ref[...] = jnp.concatenate([h[:, :128], st], axis=1)
  hhi_ref[...] = jnp.concatenate([h[:, 128:], st], axis=1)
  stab_ref[...] = st


def _tc_prep(xp, wt, b2, A):
  blk = 512
  grid = _NP // blk
  return pl.pallas_call(
      _prep_body,
      grid=(grid,),
      in_specs=[
          pl.BlockSpec((blk, _DIM), lambda i: (i, 0)),
          pl.BlockSpec((_DIM, _DIM), lambda i: (0, 0)),
          pl.BlockSpec((1, _DIM), lambda i: (0, 0)),
          pl.BlockSpec((_DIM, 16), lambda i: (0, 0)),
      ],
      out_specs=[
          pl.BlockSpec((blk, 144), lambda i: (i, 0)),
          pl.BlockSpec((blk, 144), lambda i: (i, 0)),
          pl.BlockSpec((blk, 16), lambda i: (i, 0)),
      ],
      out_shape=[
          jax.ShapeDtypeStruct((_NP, 144), jnp.float32),
          jax.ShapeDtypeStruct((_NP, 144), jnp.float32),
          jax.ShapeDtypeStruct((_NP, 16), jnp.float32),
      ],
  )(xp, wt, b2, A)


def _sc_body(hlo, hhi, stab, packedI_hbm, zeros_hbm, out_hbm,
             acc_sh, packed_v, sidx, didx, sts2, hpk2, msg2,
             alpha_v, semg0, semg1, sems0, sems1):
  cid = lax.axis_index("c")
  sid = lax.axis_index("s")
  semg = [semg0, semg1]
  sems = [sems0, sems1]

  pltpu.sync_copy(zeros_hbm, acc_sh.at[pl.ds(sid * 640, 640)])
  plsc.subcore_barrier()

  pltpu.sync_copy(packedI_hbm.at[sid], packed_v)

  iota = lax.iota(jnp.int32, 16)
  mask16 = jnp.full((16,), 0xFFFF, jnp.int32)
  sh16 = jnp.full((16,), 16, jnp.int32)

  def unpack_idx(c, s):
    for v in range(_CHUNK // 16):
      p = packed_v[c, pl.ds(v * 16, 16)]
      sidx[s, pl.ds(v * 16, 16)] = jnp.bitwise_and(p, mask16)
      didx[s, pl.ds(v * 16, 16)] = lax.shift_right_logical(p, sh16)

  def issue_gathers(s):
    pltpu.async_copy(stab.at[sidx.at[s]], sts2.at[s], semg[s])

    @pl.when(cid == 0)
    def _():
      pltpu.async_copy(hlo.at[didx.at[s]], hpk2.at[s], semg[s])

    @pl.when(cid == 1)
    def _():
      pltpu.async_copy(hhi.at[didx.at[s]], hpk2.at[s], semg[s])

  def wait_gathers(s):
    pltpu.make_async_copy(stab.at[sidx.at[s]], sts2.at[s], semg[s]).wait()

    @pl.when(cid == 0)
    def _():
      pltpu.make_async_copy(hlo.at[didx.at[s]], hpk2.at[s], semg[s]).wait()

    @pl.when(cid == 1)
    def _():
      pltpu.make_async_copy(hhi.at[didx.at[s]], hpk2.at[s], semg[s]).wait()

  unpack_idx(0, 0)
  issue_gathers(0)

  def pair_body(i, carry):
    for b in range(2):
      c = 2 * i + b
      wait_gathers(b)

      @pl.when(c >= 1)
      def _():
        pltpu.make_async_copy(
            msg2.at[1 - b], acc_sh.at[sidx.at[1 - b]], sems[1 - b]).wait()

      @pl.when(c + 1 < _NCHUNK)
      def _():
        unpack_idx(c + 1, 1 - b)
        issue_gathers(1 - b)

      for g in range(_CHUNK // 16):
        rows = iota + g * 16
        score = []
        for h in range(_HEADS):
          s_h = plsc.load_gather(
              sts2.at[b], [rows, jnp.full((16,), h, jnp.int32)])
          w = plsc.load_gather(
              hpk2.at[b], [rows, jnp.full((16,), 68 + h // 2, jnp.int32)])
          if h % 2 == 0:
            t_h = plsc.bitcast(lax.shift_left(w, sh16), jnp.float32)
          else:
            t_h = plsc.bitcast(
                jnp.bitwise_and(w, jnp.int32(-65536)), jnp.float32)
          sc = s_h + t_h
          score.append(jnp.where(sc >= 0, sc, 0.2 * sc))
        m = score[0]
        for h in range(1, _HEADS):
          m = jnp.maximum(m, score[h])
        ex = [jnp.exp(score[h] - m) for h in range(_HEADS)]
        tot = ex[0]
        for h in range(1, _HEADS):
          tot = tot + ex[h]
        inv = 1.0 / tot
        for j in range(4):
          a_j = jnp.where(cid == 0, ex[j], ex[4 + j]) * inv
          alpha_v[j, pl.ds(g * 16, 16)] = a_j

      def edge_body(k, carry2):
        kv = jnp.full((16,), k, jnp.int32)
        for v in range(4):
          w = hpk2[b, k, pl.ds(v * 16, 16)]
          lo, hi = plsc.unpack(plsc.bitcast(w, jnp.bfloat16),
                               format=plsc.PackFormat.INTERLEAVED)
          a_v = plsc.load_gather(
              alpha_v, [jnp.full((16,), v, jnp.int32), kv])
          msg2[b, k, pl.ds(v * 32, 16)] = lo * a_v
          msg2[b, k, pl.ds(v * 32 + 16, 16)] = hi * a_v
        return carry2

      lax.fori_loop(0, _CHUNK, edge_body, 0, unroll=4)

      pltpu.async_copy(msg2.at[b], acc_sh.at[sidx.at[b]], sems[b], add=True)
    return carry

  lax.fori_loop(0, _NCHUNK // 2, pair_body, 0, unroll=False)
  pltpu.make_async_copy(
      msg2.at[1], acc_sh.at[sidx.at[1]], sems[1]).wait()
  plsc.subcore_barrier()

  pltpu.sync_copy(acc_sh.at[pl.ds(sid * 640, 640)],
                  out_hbm.at[cid, pl.ds(sid * 640, 640)])


def _sc_edges(hlo, hhi, stab, packedI, zeros_hbm):
  mesh = plsc.VectorSubcoreMesh(core_axis_name="c", subcore_axis_name="s")
  kern = pl.kernel(
      _sc_body,
      out_type=jax.ShapeDtypeStruct((2, _NP, 128), jnp.float32),
      mesh=mesh,
      scratch_types=[
          pltpu.VMEM_SHARED((_NP, 128), jnp.float32),
          pltpu.VMEM((_NCHUNK, _CHUNK), jnp.int32),
          pltpu.VMEM((2, _CHUNK), jnp.int32),
          pltpu.VMEM((2, _CHUNK), jnp.int32),
          pltpu.VMEM((2, _CHUNK, 16), jnp.float32),
          pltpu.VMEM((2, _CHUNK, 72), jnp.int32),
          pltpu.VMEM((2, _CHUNK, 128), jnp.float32),
          pltpu.VMEM((4, _CHUNK), jnp.float32),
          pltpu.SemaphoreType.DMA,
          pltpu.SemaphoreType.DMA,
          pltpu.SemaphoreType.DMA,
          pltpu.SemaphoreType.DMA,
      ],
      compiler_params=pltpu.CompilerParams(
          needs_layout_passes=False, use_tc_tiling_on_sc=False),
  )
  return kern(hlo, hhi, stab, packedI, zeros_hbm)


def _post_body(acc_ref, x_ref, g_ref, be_ref, out_ref):
  acc = acc_ref[...]
  v = jnp.concatenate([acc[0], acc[1]], axis=-1) + x_ref[...]
  mean = jnp.mean(v, axis=-1, keepdims=True)
  cent = v - mean
  var = jnp.mean(cent * cent, axis=-1, keepdims=True)
  ln = cent * lax.rsqrt(var + 1e-5) * g_ref[...] + be_ref[...]
  n2 = jnp.sum(ln * ln, axis=-1, keepdims=True)
  out_ref[...] = ln * lax.rsqrt(jnp.maximum(n2, 1e-24))


def _tc_post(acc, x, g2, be2):
  blk = 1000
  grid = _N // blk
  return pl.pallas_call(
      _post_body,
      grid=(grid,),
      in_specs=[
          pl.BlockSpec((2, blk, 128), lambda i: (0, i, 0)),
          pl.BlockSpec((blk, _DIM), lambda i: (i, 0)),
          pl.BlockSpec((1, _DIM), lambda i: (0, 0)),
          pl.BlockSpec((1, _DIM), lambda i: (0, 0)),
      ],
      out_specs=pl.BlockSpec((blk, _DIM), lambda i: (i, 0)),
      out_shape=jax.ShapeDtypeStruct((_N, _DIM), jnp.float32),
  )(acc, x, g2, be2)


@jax.jit
def kernel(x, edge_index, W_weight, W_bias, a, ln_gamma, ln_beta):
  a_l = a[:, :_HD]
  a_r = a[:, _HD:]
  eye = jnp.eye(_HEADS, dtype=jnp.float32)
  A_l = (a_l[:, :, None] * eye[:, None, :]).reshape(_DIM, _HEADS)
  A_r = (a_r[:, :, None] * eye[:, None, :]).reshape(_DIM, _HEADS)
  A = jnp.concatenate([A_l, A_r], axis=1)

  xp = jnp.concatenate(
      [x, jnp.zeros((_NP - _N, _DIM), jnp.float32)], axis=0)
  wt = W_weight.T
  b2 = W_bias[None, :]

  hlo_f, hhi_f, stab = _tc_prep(xp, wt, b2, A)

  def _to_i32(tab):
    h = tab[:, :128].reshape(_NP, 4, 2, 16).transpose(0, 1, 3, 2)
    fb = jnp.concatenate(
        [h.reshape(_NP, 128), tab[:, 128:]], axis=1).astype(jnp.bfloat16)
    return jax.lax.bitcast_convert_type(fb.reshape(_NP, 72, 2), jnp.int32)

  hlo = _to_i32(hlo_f)
  hhi = _to_i32(hhi_f)

  src = edge_index[0].astype(jnp.int32)
  dst = edge_index[1].astype(jnp.int32)
  srcp = jnp.concatenate([src, jnp.full((_EPAD - _E,), _N, jnp.int32)])
  dstp = jnp.concatenate([dst, jnp.zeros((_EPAD - _E,), jnp.int32)])
  packedI = ((dstp << 16) | srcp).reshape(_TILES, _NCHUNK, _CHUNK)
  zeros_hbm = jnp.zeros((640, 128), jnp.float32)

  acc = _sc_edges(hlo, hhi, stab, packedI, zeros_hbm)

  return _tc_post(acc, x, ln_gamma[None, :], ln_beta[None, :])


# speedup vs baseline: 1.0717x; 1.0717x over previous
import jax
import jax.numpy as jnp
from jax import lax
from jax.experimental import pallas as pl
from jax.experimental.pallas import tpu as pltpu
from jax.experimental.pallas import tpu_sc as plsc

_DIM = 256
_HEADS = 8
_HD = _DIM // _HEADS
_N = 10000
_E = 160000

_NP = 10240
_TILES = 16
_CHUNK = 80
_EPT = 10240
_NCHUNK = _EPT // _CHUNK
_EPAD = _TILES * _EPT


def _prep_body(x_ref, wt_ref, b_ref, A_ref, hlo_ref, hhi_ref, stab_ref):
  h = jnp.dot(x_ref[...], wt_ref[...], preferred_element_type=jnp.float32)
  h = h + b_ref[...]
  st = jnp.dot(h, A_ref[...], preferred_element_type=jnp.float32)
  hlo_ref[...] = jnp.concatenate([h[:, :128], st], axis=1)
  hhi_ref[...] = jnp.concatenate([h[:, 128:], st], axis=1)
  stab_ref[...] = st


def _tc_prep(xp, wt, b2, A):
  blk = 512
  grid = _NP // blk
  return pl.pallas_call(
      _prep_body,
      grid=(grid,),
      in_specs=[
          pl.BlockSpec((blk, _DIM), lambda i: (i, 0)),
          pl.BlockSpec((_DIM, _DIM), lambda i: (0, 0)),
          pl.BlockSpec((1, _DIM), lambda i: (0, 0)),
          pl.BlockSpec((_DIM, 16), lambda i: (0, 0)),
      ],
      out_specs=[
          pl.BlockSpec((blk, 144), lambda i: (i, 0)),
          pl.BlockSpec((blk, 144), lambda i: (i, 0)),
          pl.BlockSpec((blk, 16), lambda i: (i, 0)),
      ],
      out_shape=[
          jax.ShapeDtypeStruct((_NP, 144), jnp.float32),
          jax.ShapeDtypeStruct((_NP, 144), jnp.float32),
          jax.ShapeDtypeStruct((_NP, 16), jnp.float32),
      ],
  )(xp, wt, b2, A)


def _sc_body(hlo, hhi, stab, packedI_hbm, zeros_hbm, out_hbm,
             acc_sh, packed_v, sidx, didx, sts2, hrow2,
             alpha_v, semg0, semg1, sems0, sems1):
  cid = lax.axis_index("c")
  sid = lax.axis_index("s")
  semg = [semg0, semg1]
  sems = [sems0, sems1]

  pltpu.sync_copy(zeros_hbm, acc_sh.at[pl.ds(sid * 640, 640)])
  plsc.subcore_barrier()

  pltpu.sync_copy(packedI_hbm.at[sid], packed_v)

  iota = lax.iota(jnp.int32, 16)
  mask16 = jnp.full((16,), 0xFFFF, jnp.int32)
  sh16 = jnp.full((16,), 16, jnp.int32)

  def unpack_idx(c, s):
    for v in range(_CHUNK // 16):
      p = packed_v[c, pl.ds(v * 16, 16)]
      sidx[s, pl.ds(v * 16, 16)] = jnp.bitwise_and(p, mask16)
      didx[s, pl.ds(v * 16, 16)] = lax.shift_right_logical(p, sh16)

  def issue_gathers(s):
    pltpu.async_copy(stab.at[sidx.at[s]], sts2.at[s], semg[s])

    @pl.when(cid == 0)
    def _():
      pltpu.async_copy(hlo.at[didx.at[s]], hrow2.at[s], semg[s])

    @pl.when(cid == 1)
    def _():
      pltpu.async_copy(hhi.at[didx.at[s]], hrow2.at[s], semg[s])

  def wait_gathers(s):
    pltpu.make_async_copy(stab.at[sidx.at[s]], sts2.at[s], semg[s]).wait()

    @pl.when(cid == 0)
    def _():
      pltpu.make_async_copy(hlo.at[didx.at[s]], hrow2.at[s], semg[s]).wait()

    @pl.when(cid == 1)
    def _():
      pltpu.make_async_copy(hhi.at[didx.at[s]], hrow2.at[s], semg[s]).wait()

  unpack_idx(0, 0)
  issue_gathers(0)

  def pair_body(i, carry):
    for b in range(2):
      c = 2 * i + b
      wait_gathers(b)

      @pl.when(c >= 1)
      def _():
        pltpu.make_async_copy(
            hrow2.at[1 - b], acc_sh.at[sidx.at[1 - b]], sems[1 - b]).wait()

      @pl.when(c + 1 < _NCHUNK)
      def _():
        unpack_idx(c + 1, 1 - b)
        issue_gathers(1 - b)

      for g in range(_CHUNK // 16):
        rows = iota + g * 16
        score = []
        for h in range(_HEADS):
          s_h = plsc.load_gather(
              sts2.at[b], [rows, jnp.full((16,), h, jnp.int32)])
          t_h = plsc.load_gather(
              hrow2.at[b], [rows, jnp.full((16,), 136 + h, jnp.int32)])
          sc = s_h + t_h
          score.append(jnp.where(sc >= 0, sc, 0.2 * sc))
        m = score[0]
        for h in range(1, _HEADS):
          m = jnp.maximum(m, score[h])
        ex = [jnp.exp(score[h] - m) for h in range(_HEADS)]
        tot = ex[0]
        for h in range(1, _HEADS):
          tot = tot + ex[h]
        inv = 1.0 / tot
        for j in range(4):
          a_j = jnp.where(cid == 0, ex[j], ex[4 + j]) * inv
          alpha_v[j, pl.ds(g * 16, 16)] = a_j

      def edge_body(k, carry2):
        kv = jnp.full((16,), k, jnp.int32)
        for j in range(4):
          a_j = plsc.load_gather(
              alpha_v, [jnp.full((16,), j, jnp.int32), kv])
          for v in range(2):
            f = (j * 2 + v) * 16
            hrow2[b, k, pl.ds(f, 16)] = hrow2[b, k, pl.ds(f, 16)] * a_j
        return carry2

      lax.fori_loop(0, _CHUNK, edge_body, 0, unroll=4)

      pltpu.async_copy(hrow2.at[b], acc_sh.at[sidx.at[b]], sems[b], add=True)
    return carry

  lax.fori_loop(0, _NCHUNK // 2, pair_body, 0, unroll=False)
  pltpu.make_async_copy(
      hrow2.at[1], acc_sh.at[sidx.at[1]], sems[1]).wait()
  plsc.subcore_barrier()

  pltpu.sync_copy(acc_sh.at[pl.ds(sid * 640, 640), pl.ds(0, 128)],
                  out_hbm.at[cid, pl.ds(sid * 640, 640)])


def _sc_edges(hlo, hhi, stab, packedI, zeros_hbm):
  mesh = plsc.VectorSubcoreMesh(core_axis_name="c", subcore_axis_name="s")
  kern = pl.kernel(
      _sc_body,
      out_type=jax.ShapeDtypeStruct((2, _NP, 128), jnp.float32),
      mesh=mesh,
      scratch_types=[
          pltpu.VMEM_SHARED((_NP, 144), jnp.float32),
          pltpu.VMEM((_NCHUNK, _CHUNK), jnp.int32),
          pltpu.VMEM((2, _CHUNK), jnp.int32),
          pltpu.VMEM((2, _CHUNK), jnp.int32),
          pltpu.VMEM((2, _CHUNK, 16), jnp.float32),
          pltpu.VMEM((2, _CHUNK, 144), jnp.float32),
          pltpu.VMEM((4, _CHUNK), jnp.float32),
          pltpu.SemaphoreType.DMA,
          pltpu.SemaphoreType.DMA,
          pltpu.SemaphoreType.DMA,
          pltpu.SemaphoreType.DMA,
      ],
      compiler_params=pltpu.CompilerParams(
          needs_layout_passes=False, use_tc_tiling_on_sc=False),
  )
  return kern(hlo, hhi, stab, packedI, zeros_hbm)


def _post_body(acc_ref, x_ref, g_ref, be_ref, out_ref):
  acc = acc_ref[...]
  v = jnp.concatenate([acc[0], acc[1]], axis=-1) + x_ref[...]
  mean = jnp.mean(v, axis=-1, keepdims=True)
  cent = v - mean
  var = jnp.mean(cent * cent, axis=-1, keepdims=True)
  ln = cent * lax.rsqrt(var + 1e-5) * g_ref[...] + be_ref[...]
  n2 = jnp.sum(ln * ln, axis=-1, keepdims=True)
  out_ref[...] = ln * lax.rsqrt(jnp.maximum(n2, 1e-24))


def _tc_post(acc, x, g2, be2):
  blk = 1000
  grid = _N // blk
  return pl.pallas_call(
      _post_body,
      grid=(grid,),
      in_specs=[
          pl.BlockSpec((2, blk, 128), lambda i: (0, i, 0)),
          pl.BlockSpec((blk, _DIM), lambda i: (i, 0)),
          pl.BlockSpec((1, _DIM), lambda i: (0, 0)),
          pl.BlockSpec((1, _DIM), lambda i: (0, 0)),
      ],
      out_specs=pl.BlockSpec((blk, _DIM), lambda i: (i, 0)),
      out_shape=jax.ShapeDtypeStruct((_N, _DIM), jnp.float32),
  )(acc, x, g2, be2)


@jax.jit
def kernel(x, edge_index, W_weight, W_bias, a, ln_gamma, ln_beta):
  a_l = a[:, :_HD]
  a_r = a[:, _HD:]
  eye = jnp.eye(_HEADS, dtype=jnp.float32)
  A_l = (a_l[:, :, None] * eye[:, None, :]).reshape(_DIM, _HEADS)
  A_r = (a_r[:, :, None] * eye[:, None, :]).reshape(_DIM, _HEADS)
  A = jnp.concatenate([A_l, A_r], axis=1)

  xp = jnp.concatenate(
      [x, jnp.zeros((_NP - _N, _DIM), jnp.float32)], axis=0)
  wt = W_weight.T
  b2 = W_bias[None, :]

  hlo, hhi, stab = _tc_prep(xp, wt, b2, A)

  src = edge_index[0].astype(jnp.int32)
  dst = edge_index[1].astype(jnp.int32)
  srcp = jnp.concatenate([src, jnp.full((_EPAD - _E,), _N, jnp.int32)])
  dstp = jnp.concatenate([dst, jnp.zeros((_EPAD - _E,), jnp.int32)])
  packedI = ((dstp << 16) | srcp).reshape(_TILES, _NCHUNK, _CHUNK)
  zeros_hbm = jnp.zeros((640, 144), jnp.float32)

  acc = _sc_edges(hlo, hhi, stab, packedI, zeros_hbm)

  return _tc_post(acc, x, ln_gamma[None, :], ln_beta[None, :])


# speedup vs baseline: 1.1190x; 1.0441x over previous
import jax
import jax.numpy as jnp
from jax import lax
from jax.experimental import pallas as pl
from jax.experimental.pallas import tpu as pltpu
from jax.experimental.pallas import tpu_sc as plsc

_DIM = 256
_HEADS = 8
_HD = _DIM // _HEADS
_N = 10000
_E = 160000

_NP = 10240
_TILES = 16
_CHUNK = 80
_EPT = 10240
_NCHUNK = _EPT // _CHUNK
_EPAD = _TILES * _EPT


def _prep_body(x_ref, wt_ref, b_ref, A_ref, hlo_ref, hhi_ref, stab_ref):
  h = jnp.dot(x_ref[...], wt_ref[...], preferred_element_type=jnp.float32)
  h = h + b_ref[...]
  st = jnp.dot(h, A_ref[...], preferred_element_type=jnp.float32)
  hlo_ref[...] = jnp.concatenate([h[:, :128], st], axis=1)
  hhi_ref[...] = jnp.concatenate([h[:, 128:], st], axis=1)
  stab_ref[...] = st


def _tc_prep(xp, wt, b2, A):
  blk = 512
  grid = _NP // blk
  return pl.pallas_call(
      _prep_body,
      grid=(grid,),
      in_specs=[
          pl.BlockSpec((blk, _DIM), lambda i: (i, 0)),
          pl.BlockSpec((_DIM, _DIM), lambda i: (0, 0)),
          pl.BlockSpec((1, _DIM), lambda i: (0, 0)),
          pl.BlockSpec((_DIM, 16), lambda i: (0, 0)),
      ],
      out_specs=[
          pl.BlockSpec((blk, 144), lambda i: (i, 0)),
          pl.BlockSpec((blk, 144), lambda i: (i, 0)),
          pl.BlockSpec((blk, 16), lambda i: (i, 0)),
      ],
      out_shape=[
          jax.ShapeDtypeStruct((_NP, 144), jnp.float32),
          jax.ShapeDtypeStruct((_NP, 144), jnp.float32),
          jax.ShapeDtypeStruct((_NP, 16), jnp.float32),
      ],
  )(xp, wt, b2, A)


def _sc_body(hlo, hhi, stab, packedI_hbm, zeros_hbm, out_hbm,
             acc_sh, packed_v, sidx, didx, sts2, hrow2,
             alpha_v, semg0, semg1, sems0, sems1):
  cid = lax.axis_index("c")
  sid = lax.axis_index("s")
  semg = [semg0, semg1]
  sems = [sems0, sems1]

  pltpu.sync_copy(zeros_hbm, acc_sh.at[pl.ds(sid * 640, 640)])
  plsc.subcore_barrier()

  pltpu.sync_copy(packedI_hbm.at[sid], packed_v)

  iota = lax.iota(jnp.int32, 16)
  mask16 = jnp.full((16,), 0xFFFF, jnp.int32)
  sh16 = jnp.full((16,), 16, jnp.int32)

  def unpack_idx(c, s):
    for v in range(_CHUNK // 16):
      p = packed_v[c, pl.ds(v * 16, 16)]
      sidx[s, pl.ds(v * 16, 16)] = jnp.bitwise_and(p, mask16)
      didx[s, pl.ds(v * 16, 16)] = lax.shift_right_logical(p, sh16)

  def issue_gathers(s):
    pltpu.async_copy(stab.at[sidx.at[s]], sts2.at[s], semg[s])

    @pl.when(cid == 0)
    def _():
      pltpu.async_copy(hlo.at[didx.at[s]], hrow2.at[s], semg[s])

    @pl.when(cid == 1)
    def _():
      pltpu.async_copy(hhi.at[didx.at[s]], hrow2.at[s], semg[s])

  def wait_gathers(s):
    pltpu.make_async_copy(stab.at[sidx.at[s]], sts2.at[s], semg[s]).wait()

    @pl.when(cid == 0)
    def _():
      pltpu.make_async_copy(hlo.at[didx.at[s]], hrow2.at[s], semg[s]).wait()

    @pl.when(cid == 1)
    def _():
      pltpu.make_async_copy(hhi.at[didx.at[s]], hrow2.at[s], semg[s]).wait()

  unpack_idx(0, 0)
  issue_gathers(0)

  def pair_body(i, carry):
    for b in range(2):
      c = 2 * i + b
      @pl.when(c >= 1)
      def _():
        pltpu.make_async_copy(
            hrow2.at[1 - b], acc_sh.at[sidx.at[1 - b]], sems[1 - b]).wait()

      @pl.when(c + 1 < _NCHUNK)
      def _():
        unpack_idx(c + 1, 1 - b)
        issue_gathers(1 - b)

      wait_gathers(b)

      for g in range(_CHUNK // 16):
        rows = iota + g * 16
        score = []
        for h in range(_HEADS):
          s_h = plsc.load_gather(
              sts2.at[b], [rows, jnp.full((16,), h, jnp.int32)])
          t_h = plsc.load_gather(
              hrow2.at[b], [rows, jnp.full((16,), 136 + h, jnp.int32)])
          sc = s_h + t_h
          score.append(jnp.where(sc >= 0, sc, 0.2 * sc))
        m = score[0]
        for h in range(1, _HEADS):
          m = jnp.maximum(m, score[h])
        ex = [jnp.exp(score[h] - m) for h in range(_HEADS)]
        tot = ex[0]
        for h in range(1, _HEADS):
          tot = tot + ex[h]
        inv = 1.0 / tot
        for j in range(4):
          a_j = jnp.where(cid == 0, ex[j], ex[4 + j]) * inv
          alpha_v[j, pl.ds(g * 16, 16)] = a_j

      def edge_body(k, carry2):
        kv = jnp.full((16,), k, jnp.int32)
        for j in range(4):
          a_j = plsc.load_gather(
              alpha_v, [jnp.full((16,), j, jnp.int32), kv])
          for v in range(2):
            f = (j * 2 + v) * 16
            hrow2[b, k, pl.ds(f, 16)] = hrow2[b, k, pl.ds(f, 16)] * a_j
        return carry2

      lax.fori_loop(0, _CHUNK, edge_body, 0, unroll=4)

      pltpu.async_copy(hrow2.at[b], acc_sh.at[sidx.at[b]], sems[b], add=True)
    return carry

  lax.fori_loop(0, _NCHUNK // 2, pair_body, 0, unroll=False)
  pltpu.make_async_copy(
      hrow2.at[1], acc_sh.at[sidx.at[1]], sems[1]).wait()
  plsc.subcore_barrier()

  pltpu.sync_copy(acc_sh.at[pl.ds(sid * 640, 640), pl.ds(0, 128)],
                  out_hbm.at[cid, pl.ds(sid * 640, 640)])


def _sc_edges(hlo, hhi, stab, packedI, zeros_hbm):
  mesh = plsc.VectorSubcoreMesh(core_axis_name="c", subcore_axis_name="s")
  kern = pl.kernel(
      _sc_body,
      out_type=jax.ShapeDtypeStruct((2, _NP, 128), jnp.float32),
      mesh=mesh,
      scratch_types=[
          pltpu.VMEM_SHARED((_NP, 144), jnp.float32),
          pltpu.VMEM((_NCHUNK, _CHUNK), jnp.int32),
          pltpu.VMEM((2, _CHUNK), jnp.int32),
          pltpu.VMEM((2, _CHUNK), jnp.int32),
          pltpu.VMEM((2, _CHUNK, 16), jnp.float32),
          pltpu.VMEM((2, _CHUNK, 144), jnp.float32),
          pltpu.VMEM((4, _CHUNK), jnp.float32),
          pltpu.SemaphoreType.DMA,
          pltpu.SemaphoreType.DMA,
          pltpu.SemaphoreType.DMA,
          pltpu.SemaphoreType.DMA,
      ],
      compiler_params=pltpu.CompilerParams(
          needs_layout_passes=False, use_tc_tiling_on_sc=False),
  )
  return kern(hlo, hhi, stab, packedI, zeros_hbm)


def _post_body(acc_ref, x_ref, g_ref, be_ref, out_ref):
  acc = acc_ref[...]
  v = jnp.concatenate([acc[0], acc[1]], axis=-1) + x_ref[...]
  mean = jnp.mean(v, axis=-1, keepdims=True)
  cent = v - mean
  var = jnp.mean(cent * cent, axis=-1, keepdims=True)
  ln = cent * lax.rsqrt(var + 1e-5) * g_ref[...] + be_ref[...]
  n2 = jnp.sum(ln * ln, axis=-1, keepdims=True)
  out_ref[...] = ln * lax.rsqrt(jnp.maximum(n2, 1e-24))


def _tc_post(acc, x, g2, be2):
  blk = 1000
  grid = _N // blk
  return pl.pallas_call(
      _post_body,
      grid=(grid,),
      in_specs=[
          pl.BlockSpec((2, blk, 128), lambda i: (0, i, 0)),
          pl.BlockSpec((blk, _DIM), lambda i: (i, 0)),
          pl.BlockSpec((1, _DIM), lambda i: (0, 0)),
          pl.BlockSpec((1, _DIM), lambda i: (0, 0)),
      ],
      out_specs=pl.BlockSpec((blk, _DIM), lambda i: (i, 0)),
      out_shape=jax.ShapeDtypeStruct((_N, _DIM), jnp.float32),
  )(acc, x, g2, be2)


@jax.jit
def kernel(x, edge_index, W_weight, W_bias, a, ln_gamma, ln_beta):
  a_l = a[:, :_HD]
  a_r = a[:, _HD:]
  eye = jnp.eye(_HEADS, dtype=jnp.float32)
  A_l = (a_l[:, :, None] * eye[:, None, :]).reshape(_DIM, _HEADS)
  A_r = (a_r[:, :, None] * eye[:, None, :]).reshape(_DIM, _HEADS)
  A = jnp.concatenate([A_l, A_r], axis=1)

  xp = jnp.concatenate(
      [x, jnp.zeros((_NP - _N, _DIM), jnp.float32)], axis=0)
  wt = W_weight.T
  b2 = W_bias[None, :]

  hlo, hhi, stab = _tc_prep(xp, wt, b2, A)

  src = edge_index[0].astype(jnp.int32)
  dst = edge_index[1].astype(jnp.int32)
  srcp = jnp.concatenate([src, jnp.full((_EPAD - _E,), _N, jnp.int32)])
  dstp = jnp.concatenate([dst, jnp.zeros((_EPAD - _E,), jnp.int32)])
  packedI = ((dstp << 16) | srcp).reshape(_TILES, _NCHUNK, _CHUNK)
  zeros_hbm = jnp.zeros((640, 144), jnp.float32)

  acc = _sc_edges(hlo, hhi, stab, packedI, zeros_hbm)

  return _tc_post(acc, x, ln_gamma[None, :], ln_beta[None, :])
